# Initial kernel scaffold; baseline (speedup 1.0000x reference)
import jax
import jax.numpy as jnp
from jax.experimental import pallas as pl


def kernel(x, Wk, Wq, Wv, Wu, bu, Wp1, bp1, Wp2, bp2, mvalues):
    raise NotImplementedError("write your pallas kernel here")


# speedup vs baseline: 61.3338x; 61.3338x over previous
import functools

import jax
import jax.numpy as jnp
import numpy as np
from jax.experimental import pallas as pl
from jax.experimental.pallas import tpu as pltpu
from jax.experimental.pallas import tpu_sc as plsc

_B, _T, _E, _H = 1, 2048, 256, 4
_K, _GADD, _NADD = 8, 2, 2
_P = _K * (2 + _GADD + _NADD)

_HIGH = jax.lax.Precision.HIGHEST

_NC, _NS, _L = 2, 16, 16
_NW = _NC * _NS
_RPT = _T // _NW
_RC = 8
_NJ = _P // _L


def _softplus(x):
    return jnp.maximum(x, 0.0) + jnp.log1p(jnp.exp(-jnp.abs(x)))


def _sigmoid(x):
    e = jnp.exp(-jnp.abs(x))
    return jnp.where(x >= 0.0, 1.0 / (1.0 + e), e / (1.0 + e))


_BM = 256


def _placement_mats():
    rfm = np.zeros((_K, _P), np.float32)
    rg = np.zeros((2 * _K, _P), np.float32)
    ro = np.zeros((2 * _K, _P), np.float32)
    onev = np.zeros((1, _P), np.float32)
    for k in range(_K):
        for j in (0, 1, 4, 5):
            rfm[k, 6 * k + j] = 1.0
        onev[0, 6 * k + 1] = 1.0
        for j in (0, 1):
            rg[2 * k + j, 6 * k + 2 + j] = 1.0
            ro[2 * k + j, 6 * k + 4 + j] = 1.0
    return rfm, rg, ro, onev


def _hyper_body(x_ref, wp1_ref, bp1_ref, wp2_ref, bp2_ref, mv_ref, g_ref,
                o_ref, rfm_ref, rg_ref, ro_ref, onev_ref,
                cols_ref, w_ref, first_ref, extra_ref):
    xb = x_ref[...]
    h1 = jnp.dot(xb, wp1_ref[...],
                 preferred_element_type=jnp.float32) + bp1_ref[...]
    h1 = jnp.maximum(h1, 0.0)
    params = jnp.dot(h1, wp2_ref[...],
                     preferred_element_type=jnp.float32) + bp2_ref[...]
    means_raw = params[:, :_K]
    sig_raw = params[:, _K:]

    t = float(_T)
    i = pl.program_id(0)
    rows = (i * _BM + jax.lax.broadcasted_iota(jnp.int32, (_BM, 1), 0)
            ).astype(jnp.float32)
    eps = 1e-4
    sc = (rows / (t - 1.0)) * (1.0 - 2.0 * eps) + eps
    diags = jnp.log(sc / (1.0 - sc))

    means = diags - _softplus(means_raw)
    means = _sigmoid(means) * (t - 1.0)
    sigmas = _softplus(sig_raw + 2.0) + 0.01

    fm = jnp.floor(means)
    g = g_ref[...]
    o = o_ref[...]
    colsf = (jnp.dot(fm, rfm_ref[...], precision=_HIGH)
             + jnp.dot(g, rg_ref[...], precision=_HIGH)
             + jnp.dot(o, ro_ref[...], precision=_HIGH)
             + onev_ref[...])
    colsf = jnp.clip(colsf, 0.0, t - 1.0)
    colsi = colsf.astype(jnp.int32)

    li = jax.lax.broadcasted_iota(jnp.int32, (_BM, _P), 1)
    dupi = jnp.zeros((_BM, _P), jnp.int32)
    extra_i = jnp.zeros((_BM, _P), jnp.int32)
    for d in range(1, _P):
        rolled = jnp.concatenate([colsi[:, _P - d:], colsi[:, :_P - d]],
                                 axis=1)
        eqd = jnp.where((colsi == rolled) & (li >= d), 1, 0)
        dupi = dupi | eqd
        eqf = jnp.concatenate([eqd[:, d:], eqd[:, :d]], axis=1)
        extra_i = extra_i + jnp.where(li < _P - d, eqf, 0)
    dup = dupi > 0
    extra = extra_i.astype(jnp.float32)

    w = jnp.zeros((_BM, _P), jnp.float32)
    for k in range(_K):
        z = (colsf - means[:, k:k + 1]) / sigmas[:, k:k + 1]
        dk = jnp.exp(-0.5 * z * z)
        dk = jnp.where(dup, 0.0, dk)
        sk = jnp.sum(dk, axis=1, keepdims=True)
        w = w + mv_ref[0, k] * (dk / sk)

    cols_ref[...] = colsi
    w_ref[...] = w
    first_ref[...] = jnp.where(dup, 0, 1).astype(jnp.int32)
    extra_ref[...] = extra


def _run_hyper(xb, Wp1, bp1, Wp2, bp2, mvalues, g, o):
    grid = (_T // _BM,)
    out_shapes = (
        jax.ShapeDtypeStruct((_T, _P), jnp.int32),
        jax.ShapeDtypeStruct((_T, _P), jnp.float32),
        jax.ShapeDtypeStruct((_T, _P), jnp.int32),
        jax.ShapeDtypeStruct((_T, _P), jnp.float32),
    )
    blk = lambda i: (i, 0)
    zero = lambda i: (0, 0)
    return pl.pallas_call(
        _hyper_body,
        grid=grid,
        in_specs=[
            pl.BlockSpec((_BM, _E), blk),
            pl.BlockSpec((_E, 4), zero),
            pl.BlockSpec((1, 4), zero),
            pl.BlockSpec((4, 2 * _K), zero),
            pl.BlockSpec((1, 2 * _K), zero),
            pl.BlockSpec((1, _K), zero),
            pl.BlockSpec((_BM, 2 * _K), blk),
            pl.BlockSpec((_BM, 2 * _K), blk),
            pl.BlockSpec((_K, _P), zero),
            pl.BlockSpec((2 * _K, _P), zero),
            pl.BlockSpec((2 * _K, _P), zero),
            pl.BlockSpec((1, _P), zero),
        ],
        out_specs=tuple(pl.BlockSpec((_BM, _P), blk) for _ in range(4)),
        out_shape=out_shapes,
    )(xb, Wp1, bp1.reshape(1, 4), Wp2, bp2.reshape(1, 2 * _K),
      mvalues.reshape(1, _K), g, o,
      *(jnp.asarray(m) for m in _placement_mats()))


def _qkt_body(x_ref, wq_ref, wk_ref, wv_ref, s_ref, v_ref):
    xb = x_ref[...]
    q = jnp.dot(xb, wq_ref[...], preferred_element_type=jnp.float32,
                precision=_HIGH)
    k = jnp.dot(xb, wk_ref[...], preferred_element_type=jnp.float32,
                precision=_HIGH)
    v_ref[0] = jnp.dot(xb, wv_ref[...], preferred_element_type=jnp.float32,
                       precision=_HIGH)
    s = jax.lax.dot_general(q, k, (((1,), (1,)), ((), ())),
                            preferred_element_type=jnp.float32,
                            precision=_HIGH)
    s_ref[0] = s * (1.0 / jnp.sqrt(float(_E)))


def _run_qkt(xb, Wq, Wk, Wv):
    return pl.pallas_call(
        _qkt_body,
        grid=(_H,),
        in_specs=[
            pl.BlockSpec((_T, _E), lambda h: (0, 0)),
            pl.BlockSpec((_E, _E), lambda h: (0, h)),
            pl.BlockSpec((_E, _E), lambda h: (0, h)),
            pl.BlockSpec((_E, _E), lambda h: (0, h)),
        ],
        out_specs=(
            pl.BlockSpec((1, _T, _T), lambda h: (h, 0, 0)),
            pl.BlockSpec((1, _T, _E), lambda h: (h, 0, 0)),
        ),
        out_shape=(
            jax.ShapeDtypeStruct((_H, _T, _T), jnp.float32),
            jax.ShapeDtypeStruct((_H, _T, _E), jnp.float32),
        ),
    )(xb, Wq, Wk, Wv)


def _sc_attend_body(s_hbm, cols_hbm, w_hbm, first_hbm, extra_hbm, a_hbm,
                    s_v, a_v, c_v, w_v, f_v, e_v):
    wid = jax.lax.axis_index("s") * _NC + jax.lax.axis_index("c")
    row_lo = wid * _RPT

    zero16 = jnp.zeros((_L,), jnp.float32)

    def zbody(n, carry):
        a_v[pl.ds(n * _L, _L)] = zero16
        return carry

    jax.lax.fori_loop(0, _RC * _T // _L, zbody, 0)

    def head_loop(h, carry0):
        def chunk_loop(ci, carry1):
            row0 = row_lo + ci * _RC
            s_off = (h * _T + row0) * _T
            m_off = row0 * _P
            pltpu.sync_copy(s_hbm.at[pl.ds(s_off, _RC * _T)], s_v)
            pltpu.sync_copy(cols_hbm.at[pl.ds(m_off, _RC * _P)], c_v)
            pltpu.sync_copy(w_hbm.at[pl.ds(m_off, _RC * _P)], w_v)
            pltpu.sync_copy(first_hbm.at[pl.ds(m_off, _RC * _P)], f_v)
            pltpu.sync_copy(extra_hbm.at[pl.ds(m_off, _RC * _P)], e_v)

            def row_loop(r, carry2):
                row = row0 + r
                base = r * _P
                rowt = r * _T
                cs = []
                vs = []
                for j in range(_NJ):
                    c = c_v[pl.ds(base + j * _L, _L)]
                    wv = w_v[pl.ds(base + j * _L, _L)]
                    d = plsc.load_gather(s_v, [rowt + c])
                    d = jnp.where(c > row, 0.0, d)
                    cs.append(c)
                    vs.append(wv * d)
                vmax = jnp.maximum(jnp.maximum(vs[0], vs[1]), vs[2])
                mx = jnp.max(vmax)
                es = [jnp.exp(v - mx) for v in vs]
                ssum = jnp.sum(es[0] + es[1] + es[2])
                e0 = jnp.exp(zero16 - mx)
                for j in range(_NJ):
                    fm = f_v[pl.ds(base + j * _L, _L)] > 0
                    ex = e_v[pl.ds(base + j * _L, _L)]
                    val = (es[j] + ex * e0) / ssum
                    plsc.store_scatter(a_v, [rowt + cs[j]], val, mask=fm)
                return carry2

            jax.lax.fori_loop(0, _RC, row_loop, 0)
            pltpu.sync_copy(a_v, a_hbm.at[pl.ds(s_off, _RC * _T)])

            def rezero(r, carry2):
                base = r * _P
                rowt = r * _T
                for j in range(_NJ):
                    c = c_v[pl.ds(base + j * _L, _L)]
                    fm = f_v[pl.ds(base + j * _L, _L)] > 0
                    plsc.store_scatter(a_v, [rowt + c], zero16, mask=fm)
                return carry2

            jax.lax.fori_loop(0, _RC, rezero, 0)
            return carry1

        jax.lax.fori_loop(0, _RPT // _RC, chunk_loop, 0)
        return carry0

    jax.lax.fori_loop(0, _H, head_loop, 0)


@functools.lru_cache(maxsize=1)
def _build_sc_attend():
    return pl.kernel(
        _sc_attend_body,
        out_type=jax.ShapeDtypeStruct((_H * _T * _T,), jnp.float32),
        mesh=plsc.VectorSubcoreMesh(core_axis_name="c", subcore_axis_name="s",
                                    num_cores=_NC, num_subcores=_NS),
        scratch_types=[
            pltpu.VMEM((_RC * _T,), jnp.float32),
            pltpu.VMEM((_RC * _T,), jnp.float32),
            pltpu.VMEM((_RC * _P,), jnp.int32),
            pltpu.VMEM((_RC * _P,), jnp.float32),
            pltpu.VMEM((_RC * _P,), jnp.int32),
            pltpu.VMEM((_RC * _P,), jnp.float32),
        ],
        compiler_params=pltpu.CompilerParams(needs_layout_passes=False),
    )


_BM2 = 512


def _out_body(a_ref, v_ref, wu_ref, bu_ref, o_ref):
    h = pl.program_id(1)
    y = jnp.dot(a_ref[0], v_ref[0], preferred_element_type=jnp.float32,
                precision=_HIGH)
    contrib = jnp.dot(y, wu_ref[...], preferred_element_type=jnp.float32,
                      precision=_HIGH)

    @pl.when(h == 0)
    def _():
        o_ref[...] = contrib + bu_ref[...]

    @pl.when(h != 0)
    def _():
        o_ref[...] = o_ref[...] + contrib


def _run_out(a, v, Wu, bu):
    return pl.pallas_call(
        _out_body,
        grid=(_T // _BM2, _H),
        in_specs=[
            pl.BlockSpec((1, _BM2, _T), lambda i, h: (h, i, 0)),
            pl.BlockSpec((1, _T, _E), lambda i, h: (h, 0, 0)),
            pl.BlockSpec((_E, _E), lambda i, h: (h, 0)),
            pl.BlockSpec((1, _E), lambda i, h: (0, 0)),
        ],
        out_specs=pl.BlockSpec((_BM2, _E), lambda i, h: (i, 0)),
        out_shape=jax.ShapeDtypeStruct((_T, _E), jnp.float32),
    )(a, v, Wu, bu.reshape(1, _E))


def kernel(x, Wk, Wq, Wv, Wu, bu, Wp1, bp1, Wp2, bp2, mvalues):
    xb = x.reshape(_T, _E)
    k1, k2 = jax.random.split(jax.random.key(42))
    gints = jax.random.randint(k1, (_B, _T, _K, _GADD, 1), 0, _T)
    offs = jax.random.randint(k2, (_B, _T, _K, _NADD, 1), -2, 3)
    g = gints.astype(jnp.float32).reshape(_T, _K * _GADD)
    o = offs.astype(jnp.float32).reshape(_T, _K * _NADD)

    cols, w, first, extra = _run_hyper(xb, Wp1, bp1, Wp2, bp2, mvalues, g, o)
    s, v = _run_qkt(xb, Wq, Wk, Wv)
    a = _build_sc_attend()(s.reshape(-1), cols.reshape(-1), w.reshape(-1),
                           first.reshape(-1), extra.reshape(-1))
    out = _run_out(a.reshape(_H, _T, _T), v, Wu, bu)
    return out.reshape(_B, _T, _E)


# speedup vs baseline: 78.7388x; 1.2838x over previous
import functools

import jax
import jax.numpy as jnp
import numpy as np
from jax.experimental import pallas as pl
from jax.experimental.pallas import tpu as pltpu
from jax.experimental.pallas import tpu_sc as plsc

_B, _T, _E, _H = 1, 2048, 256, 4
_K, _GADD, _NADD = 8, 2, 2
_P = _K * (2 + _GADD + _NADD)

_HIGH = jax.lax.Precision.HIGHEST

_NC, _NS, _L = 2, 16, 16
_NW = _NC * _NS
_RPT = _T // _NW
_RC = 8
_NJ = _P // _L


def _softplus(x):
    return jnp.maximum(x, 0.0) + jnp.log1p(jnp.exp(-jnp.abs(x)))


def _sigmoid(x):
    e = jnp.exp(-jnp.abs(x))
    return jnp.where(x >= 0.0, 1.0 / (1.0 + e), e / (1.0 + e))


_BM = 256


def _placement_mats():
    rfm = np.zeros((_K, _P), np.float32)
    rg = np.zeros((2 * _K, _P), np.float32)
    ro = np.zeros((2 * _K, _P), np.float32)
    onev = np.zeros((1, _P), np.float32)
    for k in range(_K):
        for j in (0, 1, 4, 5):
            rfm[k, 6 * k + j] = 1.0
        onev[0, 6 * k + 1] = 1.0
        for j in (0, 1):
            rg[2 * k + j, 6 * k + 2 + j] = 1.0
            ro[2 * k + j, 6 * k + 4 + j] = 1.0
    return rfm, rg, ro, onev


def _hyper_body(x_ref, wp1_ref, bp1_ref, wp2_ref, bp2_ref, mv_ref, g_ref,
                o_ref, rfm_ref, rg_ref, ro_ref, onev_ref,
                cols_ref, w_ref, first_ref, extra_ref):
    xb = x_ref[...]
    h1 = jnp.dot(xb, wp1_ref[...],
                 preferred_element_type=jnp.float32) + bp1_ref[...]
    h1 = jnp.maximum(h1, 0.0)
    params = jnp.dot(h1, wp2_ref[...],
                     preferred_element_type=jnp.float32) + bp2_ref[...]
    means_raw = params[:, :_K]
    sig_raw = params[:, _K:]

    t = float(_T)
    i = pl.program_id(0)
    rows = (i * _BM + jax.lax.broadcasted_iota(jnp.int32, (_BM, 1), 0)
            ).astype(jnp.float32)
    eps = 1e-4
    sc = (rows / (t - 1.0)) * (1.0 - 2.0 * eps) + eps
    diags = jnp.log(sc / (1.0 - sc))

    means = diags - _softplus(means_raw)
    means = _sigmoid(means) * (t - 1.0)
    sigmas = _softplus(sig_raw + 2.0) + 0.01

    fm = jnp.floor(means)
    g = g_ref[...]
    o = o_ref[...]
    colsf = (jnp.dot(fm, rfm_ref[...], precision=_HIGH)
             + jnp.dot(g, rg_ref[...], precision=_HIGH)
             + jnp.dot(o, ro_ref[...], precision=_HIGH)
             + onev_ref[...])
    colsf = jnp.clip(colsf, 0.0, t - 1.0)
    colsi = colsf.astype(jnp.int32)

    li = jax.lax.broadcasted_iota(jnp.int32, (_BM, _P), 1)
    dupi = jnp.zeros((_BM, _P), jnp.int32)
    extra_i = jnp.zeros((_BM, _P), jnp.int32)
    for d in range(1, _P):
        rolled = jnp.concatenate([colsi[:, _P - d:], colsi[:, :_P - d]],
                                 axis=1)
        eqd = jnp.where((colsi == rolled) & (li >= d), 1, 0)
        dupi = dupi | eqd
        eqf = jnp.concatenate([eqd[:, d:], eqd[:, :d]], axis=1)
        extra_i = extra_i + jnp.where(li < _P - d, eqf, 0)
    dup = dupi > 0
    extra = extra_i.astype(jnp.float32)

    w = jnp.zeros((_BM, _P), jnp.float32)
    for k in range(_K):
        z = (colsf - means[:, k:k + 1]) / sigmas[:, k:k + 1]
        dk = jnp.exp(-0.5 * z * z)
        dk = jnp.where(dup, 0.0, dk)
        sk = jnp.sum(dk, axis=1, keepdims=True)
        w = w + mv_ref[0, k] * (dk / sk)

    cols_ref[...] = colsi
    w_ref[...] = w
    first_ref[...] = jnp.where(dup, 0, 1).astype(jnp.int32)
    extra_ref[...] = extra


def _run_hyper(xb, Wp1, bp1, Wp2, bp2, mvalues, g, o):
    grid = (_T // _BM,)
    out_shapes = (
        jax.ShapeDtypeStruct((_T, _P), jnp.int32),
        jax.ShapeDtypeStruct((_T, _P), jnp.float32),
        jax.ShapeDtypeStruct((_T, _P), jnp.int32),
        jax.ShapeDtypeStruct((_T, _P), jnp.float32),
    )
    blk = lambda i: (i, 0)
    zero = lambda i: (0, 0)
    return pl.pallas_call(
        _hyper_body,
        grid=grid,
        in_specs=[
            pl.BlockSpec((_BM, _E), blk),
            pl.BlockSpec((_E, 4), zero),
            pl.BlockSpec((1, 4), zero),
            pl.BlockSpec((4, 2 * _K), zero),
            pl.BlockSpec((1, 2 * _K), zero),
            pl.BlockSpec((1, _K), zero),
            pl.BlockSpec((_BM, 2 * _K), blk),
            pl.BlockSpec((_BM, 2 * _K), blk),
            pl.BlockSpec((_K, _P), zero),
            pl.BlockSpec((2 * _K, _P), zero),
            pl.BlockSpec((2 * _K, _P), zero),
            pl.BlockSpec((1, _P), zero),
        ],
        out_specs=tuple(pl.BlockSpec((_BM, _P), blk) for _ in range(4)),
        out_shape=out_shapes,
    )(xb, Wp1, bp1.reshape(1, 4), Wp2, bp2.reshape(1, 2 * _K),
      mvalues.reshape(1, _K), g, o,
      *(jnp.asarray(m) for m in _placement_mats()))


def _qkt_body(x_ref, wq_ref, wk_ref, wv_ref, s_ref, v_ref):
    xb = x_ref[...]
    q = jnp.dot(xb, wq_ref[...], preferred_element_type=jnp.float32)
    k = jnp.dot(xb, wk_ref[...], preferred_element_type=jnp.float32)
    v_ref[0] = jnp.dot(xb, wv_ref[...], preferred_element_type=jnp.float32)
    s = jax.lax.dot_general(q, k, (((1,), (1,)), ((), ())),
                            preferred_element_type=jnp.float32,
                            precision=_HIGH)
    s_ref[0] = s * (1.0 / jnp.sqrt(float(_E)))


def _run_qkt(xb, Wq, Wk, Wv):
    return pl.pallas_call(
        _qkt_body,
        grid=(_H,),
        in_specs=[
            pl.BlockSpec((_T, _E), lambda h: (0, 0)),
            pl.BlockSpec((_E, _E), lambda h: (0, h)),
            pl.BlockSpec((_E, _E), lambda h: (0, h)),
            pl.BlockSpec((_E, _E), lambda h: (0, h)),
        ],
        out_specs=(
            pl.BlockSpec((1, _T, _T), lambda h: (h, 0, 0)),
            pl.BlockSpec((1, _T, _E), lambda h: (h, 0, 0)),
        ),
        out_shape=(
            jax.ShapeDtypeStruct((_H, _T, _T), jnp.float32),
            jax.ShapeDtypeStruct((_H, _T, _E), jnp.float32),
        ),
    )(xb, Wq, Wk, Wv)


def _sc_attend_body(s_hbm, cols_hbm, w_hbm, first_hbm, extra_hbm, a_hbm,
                    s_v, a_v, c_v, w_v, f_v, e_v):
    wid = jax.lax.axis_index("s") * _NC + jax.lax.axis_index("c")
    row_lo = wid * _RPT

    zero16 = jnp.zeros((_L,), jnp.float32)

    def zbody(n, carry):
        def zrow(r, c2):
            a_v[r, pl.ds(n * _L, _L)] = zero16
            return c2
        return jax.lax.fori_loop(0, _RC, zrow, carry)

    jax.lax.fori_loop(0, _T // _L, zbody, 0)

    def head_loop(h, carry0):
        def chunk_loop(ci, carry1):
            row0 = row_lo + ci * _RC
            pltpu.sync_copy(s_hbm.at[h, pl.ds(row0, _RC)], s_v)
            pltpu.sync_copy(cols_hbm.at[pl.ds(row0, _RC)], c_v)
            pltpu.sync_copy(w_hbm.at[pl.ds(row0, _RC)], w_v)
            pltpu.sync_copy(first_hbm.at[pl.ds(row0, _RC)], f_v)
            pltpu.sync_copy(extra_hbm.at[pl.ds(row0, _RC)], e_v)

            def row_loop(r, carry2):
                row = row0 + r
                rvec = jnp.zeros((_L,), jnp.int32) + r
                cs = []
                vs = []
                for j in range(_NJ):
                    c = c_v[r, pl.ds(j * _L, _L)]
                    wv = w_v[r, pl.ds(j * _L, _L)]
                    d = plsc.load_gather(s_v, [rvec, c])
                    d = jnp.where(c > row, 0.0, d)
                    cs.append(c)
                    vs.append(wv * d)
                vmax = jnp.maximum(jnp.maximum(vs[0], vs[1]), vs[2])
                mx = jnp.max(vmax)
                es = [jnp.exp(v - mx) for v in vs]
                ssum = jnp.sum(es[0] + es[1] + es[2])
                e0 = jnp.exp(zero16 - mx)
                for j in range(_NJ):
                    fm = f_v[r, pl.ds(j * _L, _L)] > 0
                    ex = e_v[r, pl.ds(j * _L, _L)]
                    val = (es[j] + ex * e0) / ssum
                    plsc.store_scatter(a_v, [rvec, cs[j]], val, mask=fm)
                return carry2

            jax.lax.fori_loop(0, _RC, row_loop, 0)
            pltpu.sync_copy(a_v, a_hbm.at[h, pl.ds(row0, _RC)])

            def rezero(r, carry2):
                rvec = jnp.zeros((_L,), jnp.int32) + r
                for j in range(_NJ):
                    c = c_v[r, pl.ds(j * _L, _L)]
                    fm = f_v[r, pl.ds(j * _L, _L)] > 0
                    plsc.store_scatter(a_v, [rvec, c], zero16, mask=fm)
                return carry2

            jax.lax.fori_loop(0, _RC, rezero, 0)
            return carry1

        jax.lax.fori_loop(0, _RPT // _RC, chunk_loop, 0)
        return carry0

    jax.lax.fori_loop(0, _H, head_loop, 0)


@functools.lru_cache(maxsize=1)
def _build_sc_attend():
    return pl.kernel(
        _sc_attend_body,
        out_type=jax.ShapeDtypeStruct((_H, _T, _T), jnp.float32),
        mesh=plsc.VectorSubcoreMesh(core_axis_name="c", subcore_axis_name="s",
                                    num_cores=_NC, num_subcores=_NS),
        scratch_types=[
            pltpu.VMEM((_RC, _T), jnp.float32),
            pltpu.VMEM((_RC, _T), jnp.float32),
            pltpu.VMEM((_RC, _P), jnp.int32),
            pltpu.VMEM((_RC, _P), jnp.float32),
            pltpu.VMEM((_RC, _P), jnp.int32),
            pltpu.VMEM((_RC, _P), jnp.float32),
        ],
        compiler_params=pltpu.CompilerParams(needs_layout_passes=False),
    )


_BM2 = 512


def _out_body(a_ref, v_ref, wu_ref, bu_ref, o_ref):
    h = pl.program_id(1)
    y = jnp.dot(a_ref[0], v_ref[0], preferred_element_type=jnp.float32,
                precision=_HIGH)
    contrib = jnp.dot(y, wu_ref[...], preferred_element_type=jnp.float32)

    @pl.when(h == 0)
    def _():
        o_ref[...] = contrib + bu_ref[...]

    @pl.when(h != 0)
    def _():
        o_ref[...] = o_ref[...] + contrib


def _run_out(a, v, Wu, bu):
    return pl.pallas_call(
        _out_body,
        grid=(_T // _BM2, _H),
        in_specs=[
            pl.BlockSpec((1, _BM2, _T), lambda i, h: (h, i, 0)),
            pl.BlockSpec((1, _T, _E), lambda i, h: (h, 0, 0)),
            pl.BlockSpec((_E, _E), lambda i, h: (h, 0)),
            pl.BlockSpec((1, _E), lambda i, h: (0, 0)),
        ],
        out_specs=pl.BlockSpec((_BM2, _E), lambda i, h: (i, 0)),
        out_shape=jax.ShapeDtypeStruct((_T, _E), jnp.float32),
    )(a, v, Wu, bu.reshape(1, _E))


def kernel(x, Wk, Wq, Wv, Wu, bu, Wp1, bp1, Wp2, bp2, mvalues):
    xb = x.reshape(_T, _E)
    k1, k2 = jax.random.split(jax.random.key(42))
    gints = jax.random.randint(k1, (_B, _T, _K, _GADD, 1), 0, _T)
    offs = jax.random.randint(k2, (_B, _T, _K, _NADD, 1), -2, 3)
    g = gints.astype(jnp.float32).reshape(_T, _K * _GADD)
    o = offs.astype(jnp.float32).reshape(_T, _K * _NADD)

    cols, w, first, extra = _run_hyper(xb, Wp1, bp1, Wp2, bp2, mvalues, g, o)
    s, v = _run_qkt(xb, Wq, Wk, Wv)
    a = _build_sc_attend()(s, cols, w, first, extra)
    out = _run_out(a, v, Wu, bu)
    return out.reshape(_B, _T, _E)
